# Initial kernel scaffold; baseline (speedup 1.0000x reference)
import jax
import jax.numpy as jnp
from jax.experimental import pallas as pl


def kernel(joint_features, topology_features, joint_mask, edge_mask, bone_name_tokens, source_indices, target_indices, edge_direction, params):
    raise NotImplementedError("write your pallas kernel here")


# speedup vs baseline: 3.1364x; 3.1364x over previous
import jax
import jax.numpy as jnp
from jax import lax
from jax.experimental import pallas as pl
from jax.experimental.pallas import tpu as pltpu

_B, _J, _E, _EP = 256, 64, 126, 128
_NODE, _EDGE, _FFN, _LAYERS = 128, 32, 2048, 4
_IN, _TOPO, _VOCAB, _TOK, _CEMB, _CONV, _BONE = 9, 6, 64, 32, 32, 64, 64
_BB = 8
_NW_LAYER = 20


def _lnorm(x, g, b):
    m = jnp.mean(x, axis=-1, keepdims=True)
    v = jnp.mean((x - m) ** 2, axis=-1, keepdims=True)
    return (x - m) / jnp.sqrt(v + 1e-5) * g + b


def _body(*refs):
    jf_r, topo_r, jm_r, tok_r, idxc_r, idxr_r, emk_r = refs[:7]
    wr = refs[7:-2]
    rot_r, conf_r = refs[-2:]
    f32, bf16 = jnp.float32, jnp.bfloat16
    R = _J * _BB
    RT = R * _TOK

    def dot(x, w):
        return jnp.dot(x.astype(bf16), w[...], preferred_element_type=f32)

    tokc = tok_r[...]
    oh = (tokc == lax.broadcasted_iota(jnp.int32, (RT, _VOCAB), 1)).astype(bf16)
    emb = jnp.dot(oh, wr[2][...], preferred_element_type=f32)
    zrow = jnp.zeros((1, _CEMB), f32)
    tpos = lax.broadcasted_iota(jnp.int32, (RT, _CEMB), 0) % _TOK
    embm = jnp.where(tpos == 0, 0.0,
                     jnp.concatenate([zrow, emb[:-1, :]], axis=0))
    embp = jnp.where(tpos == _TOK - 1, 0.0,
                     jnp.concatenate([emb[1:, :], zrow], axis=0))
    conv = dot(embm, wr[3]) + dot(emb, wr[4]) + dot(embp, wr[5]) + wr[6][...]
    pooled = jnp.max(jax.nn.relu(conv).reshape(R, _TOK, _CONV), axis=1)
    h = jnp.concatenate([pooled, topo_r[...].reshape(R, _TOPO)], axis=1)
    bone = jax.nn.relu(dot(h, wr[7]) + wr[8][...])
    node = (dot(jf_r[...].reshape(R, _IN), wr[0]) + wr[1][...]
            + dot(bone, wr[9]) + wr[10][...])

    idxc = idxc_r[...]
    idxr = idxr_r[...]
    iotaJ = lax.broadcasted_iota(jnp.int32, (_EP, _J), 1)
    S = (idxc[:, 0:1] == iotaJ).astype(bf16)
    T = (idxc[:, 1:2] == iotaJ).astype(bf16)
    Tt = (idxr[1:2, :] ==
          lax.broadcasted_iota(jnp.int32, (_J, _EP), 0)).astype(bf16)
    D = (idxc[:, 2:3] == lax.broadcasted_iota(jnp.int32, (_EP, 8), 1)).astype(bf16)
    eattr0 = jnp.dot(D, wr[11][...], preferred_element_type=f32)
    eattr_r = jnp.concatenate([eattr0] * _BB, axis=1).reshape(_EP * _BB, _NODE)
    emask = emk_r[...][:, 0:1]
    cntJ = jnp.maximum(jnp.sum(Tt.astype(f32), axis=1, keepdims=True), 1.0)

    for l in range(_LAYERS):
        b = 12 + l * _NW_LAYER
        normed = _lnorm(node, wr[b + 12][...], wr[b + 13][...])
        nb = normed.astype(bf16).reshape(_J, _BB * _NODE)
        srcg = jnp.dot(S, nb, preferred_element_type=f32).reshape(_EP * _BB, _NODE)
        tgtg = jnp.dot(T, nb, preferred_element_type=f32).reshape(_EP * _BB, _NODE)
        comb = jnp.concatenate([srcg, tgtg, eattr_r], axis=1)
        m1 = jax.nn.relu(dot(comb, wr[b]) + wr[b + 1][...])
        msgs = (dot(m1, wr[b + 2]) + wr[b + 3][...])
        msgs_w = msgs.reshape(_EP, _BB * _NODE) * emask
        agg_w = jnp.dot(Tt, msgs_w.astype(bf16),
                        preferred_element_type=f32) / cntJ
        proj = dot(agg_w.reshape(R, _NODE), wr[b + 4]) + wr[b + 5][...]
        comb2 = jnp.concatenate([normed, proj], axis=1)
        ug = jax.nn.sigmoid(dot(comb2, wr[b + 8]) + wr[b + 9][...])
        rg = jax.nn.sigmoid(dot(comb2, wr[b + 6]) + wr[b + 7][...])
        cc = jnp.tanh(dot(jnp.concatenate([rg * normed, proj], axis=1), wr[b + 10])
                      + wr[b + 11][...])
        node = node + (1.0 - ug) * normed + ug * cc
        n2 = _lnorm(node, wr[b + 14][...], wr[b + 15][...])
        ffp = dot(n2, wr[b + 16]) + wr[b + 17][...]
        ffh = 0.5 * ffp * (1.0 + lax.erf(ffp * 0.7071067811865476))
        node = node + dot(ffh, wr[b + 18]) + wr[b + 19][...]
        eattr_r = msgs_w.reshape(_EP * _BB, _NODE)

    e = 12 + _LAYERS * _NW_LAYER
    out = _lnorm(node, wr[e][...], wr[e + 1][...])
    o8 = dot(out, wr[e + 2]) + wr[e + 3][...]
    raw = o8[:, 0:4]
    nrm = jnp.maximum(jnp.sqrt(jnp.sum(raw * raw, axis=1, keepdims=True)), 1e-8)
    m = jm_r[...].reshape(R, 1)
    rot_r[...] = ((raw / nrm) * m).reshape(_J, _BB, 4)
    conf_r[...] = (jax.nn.sigmoid(o8[:, 4:5]) * m).reshape(_J, _BB, 1)


def kernel(joint_features, topology_features, joint_mask, edge_mask,
           bone_name_tokens, source_indices, target_indices, edge_direction, params):
    f32, bf16 = jnp.float32, jnp.bfloat16
    G = _B // _BB
    jf = joint_features.transpose(1, 0, 2)
    topo = topology_features.transpose(1, 0, 2)
    jm = joint_mask.transpose(1, 0)[:, :, None]
    tok = (bone_name_tokens.astype(jnp.int32).transpose(1, 0, 2)
           .reshape(_J, G, _BB, _TOK).transpose(1, 0, 2, 3)
           .reshape(G * _J * _BB * _TOK, 1))
    idxc = jnp.full((_EP, 8), _J, jnp.int32)
    idxc = idxc.at[:_E, 0].set(source_indices.astype(jnp.int32))
    idxc = idxc.at[:_E, 1].set(target_indices.astype(jnp.int32))
    idxc = idxc.at[:, 2].set(0).at[:_E, 2].set(edge_direction.astype(jnp.int32))
    idxr = idxc.T
    emk = jnp.zeros((_EP, 8), f32).at[:_E, 0].set(edge_mask.astype(f32))

    p = params

    def w2(d):
        return d["w"].astype(bf16)

    def b2(d):
        return d["b"].reshape(1, -1).astype(f32)

    ws = [
        w2(p["input_proj"]), b2(p["input_proj"]),
        p["char_embed"].astype(bf16),
        p["conv_w"][0].astype(bf16), p["conv_w"][1].astype(bf16),
        p["conv_w"][2].astype(bf16),
        p["conv_b"].reshape(1, _CONV).astype(f32),
        w2(p["bone_out"]), b2(p["bone_out"]),
        w2(p["bone_proj"]), b2(p["bone_proj"]),
        jnp.zeros((8, _NODE), bf16).at[:2, :_EDGE].set(p["edge_dir_embed"].astype(bf16)),
    ]
    for bp in p["blocks"]:
        em1p = jnp.zeros((3 * _NODE, 4 * _EDGE), f32).at[:2 * _NODE + _EDGE].set(
            bp["edge_mlp1"]["w"])
        em2p = jnp.zeros((4 * _EDGE, _NODE), f32).at[:, :_EDGE].set(
            bp["edge_mlp2"]["w"])
        em2bp = jnp.zeros((1, _NODE), f32).at[0, :_EDGE].set(bp["edge_mlp2"]["b"])
        mpp = jnp.zeros((_NODE, _NODE), f32).at[:_EDGE].set(bp["msg_proj"]["w"])
        ws += [
            em1p.astype(bf16), b2(bp["edge_mlp1"]),
            em2p.astype(bf16), em2bp,
            mpp.astype(bf16), b2(bp["msg_proj"]),
            w2(bp["reset"]), b2(bp["reset"]),
            w2(bp["update"]), b2(bp["update"]),
            w2(bp["cand"]), b2(bp["cand"]),
            bp["norm1"]["g"].reshape(1, _NODE), bp["norm1"]["b"].reshape(1, _NODE),
            bp["norm2"]["g"].reshape(1, _NODE), bp["norm2"]["b"].reshape(1, _NODE),
            w2(bp["ffn1"]), b2(bp["ffn1"]),
            w2(bp["ffn2"]), b2(bp["ffn2"]),
        ]
    dc_w = (jnp.zeros((_NODE, 8), f32)
            .at[:, 0:4].set(p["delta"]["w"]).at[:, 4:5].set(p["conf"]["w"])).astype(bf16)
    dc_b = (jnp.zeros((1, 8), f32)
            .at[0, 0:4].set(p["delta"]["b"]).at[0, 4:5].set(p["conf"]["b"]))
    ws += [p["out_norm"]["g"].reshape(1, _NODE), p["out_norm"]["b"].reshape(1, _NODE),
           dc_w, dc_b]

    def _const(i):
        return (0, 0)

    in_specs = [
        pl.BlockSpec((_J, _BB, _IN), lambda i: (0, i, 0)),
        pl.BlockSpec((_J, _BB, _TOPO), lambda i: (0, i, 0)),
        pl.BlockSpec((_J, _BB, 1), lambda i: (0, i, 0)),
        pl.BlockSpec((_J * _BB * _TOK, 1), lambda i: (i, 0)),
        pl.BlockSpec((_EP, 8), _const),
        pl.BlockSpec((8, _EP), _const),
        pl.BlockSpec((_EP, 8), _const),
    ] + [pl.BlockSpec(w.shape, _const) for w in ws]

    rot, conf = pl.pallas_call(
        _body,
        grid=(G,),
        in_specs=in_specs,
        out_specs=[pl.BlockSpec((_J, _BB, 4), lambda i: (0, i, 0)),
                   pl.BlockSpec((_J, _BB, 1), lambda i: (0, i, 0))],
        out_shape=[jax.ShapeDtypeStruct((_J, _B, 4), f32),
                   jax.ShapeDtypeStruct((_J, _B, 1), f32)],
        compiler_params=pltpu.CompilerParams(dimension_semantics=("arbitrary",)),
    )(jf, topo, jm, tok, idxc, idxr, emk, *ws)
    return rot.transpose(1, 0, 2), conf.transpose(1, 0, 2)


# speedup vs baseline: 4.1528x; 1.3241x over previous
import jax
import jax.numpy as jnp
from jax import lax
from jax.experimental import pallas as pl
from jax.experimental.pallas import tpu as pltpu

_B, _J, _E, _EP = 256, 64, 126, 128
_NODE, _EDGE, _FFN, _LAYERS = 128, 32, 2048, 4
_IN, _TOPO, _VOCAB, _TOK, _CEMB, _CONV, _BONE = 9, 6, 64, 32, 32, 64, 64
_BB = 8
_NW_LAYER = 20
_NW_PRE = 11


def _lnorm(x, g, b):
    m = jnp.mean(x, axis=-1, keepdims=True)
    v = jnp.mean((x - m) ** 2, axis=-1, keepdims=True)
    return (x - m) / jnp.sqrt(v + 1e-5) * g + b


def _body(*refs):
    jf_r, topo_r, jm_r, tok_r, lane_r, idxc_r, idxr_r, emk_r = refs[:8]
    wr = refs[8:-2]
    rot_r, conf_r = refs[-2:]
    f32, bf16 = jnp.float32, jnp.bfloat16
    R = _J * _BB
    R2 = R // 2
    RT2 = R2 * _TOK

    def dot(x, w):
        return jnp.dot(x.astype(bf16), w[...], preferred_element_type=f32)

    tokp = tok_r[...]
    bc = jnp.dot(tokp, wr[2][...], preferred_element_type=f32)
    oh2 = (bc == lane_r[...]).astype(bf16)
    z2 = jnp.zeros((R2, _NODE), bf16)
    ohm = jnp.concatenate([z2, oh2[:-R2]], axis=0)
    ohp = jnp.concatenate([oh2[R2:], z2], axis=0)
    conv = (jnp.dot(ohm, wr[3][...], preferred_element_type=f32)
            + jnp.dot(oh2, wr[4][...], preferred_element_type=f32)
            + jnp.dot(ohp, wr[5][...], preferred_element_type=f32) + wr[6][...])
    pooled = jnp.max(jax.nn.relu(conv).reshape(_TOK, R2, _NODE), axis=0)
    h = jnp.concatenate([pooled, topo_r[...]], axis=1)
    bone = jax.nn.relu(dot(h, wr[7]) + wr[8][...])
    node2 = dot(jf_r[...], wr[0]) + wr[1][...] + dot(bone, wr[9])
    node = node2.reshape(R, _NODE)

    idxc = idxc_r[...]
    idxr = idxr_r[...]
    iotaJ = lax.broadcasted_iota(jnp.int32, (_EP, _J), 1)
    S = (idxc[:, 0:1] == iotaJ).astype(bf16)
    T = (idxc[:, 1:2] == iotaJ).astype(bf16)
    Tt = (idxr[1:2, :] ==
          lax.broadcasted_iota(jnp.int32, (_J, _EP), 0)).astype(bf16)
    D = (idxc[:, 2:3] == lax.broadcasted_iota(jnp.int32, (_EP, 8), 1)).astype(bf16)
    eattr0 = jnp.dot(D, wr[10][...], preferred_element_type=f32)
    eattr_r = jnp.concatenate([eattr0] * _BB, axis=1).reshape(_EP * _BB, _NODE)
    emask = emk_r[...][:, 0:1]
    cntJ = jnp.maximum(jnp.sum(Tt.astype(f32), axis=1, keepdims=True), 1.0)

    for l in range(_LAYERS):
        b = _NW_PRE + l * _NW_LAYER
        normed = _lnorm(node, wr[b + 12][...], wr[b + 13][...])
        nb = normed.astype(bf16).reshape(_J, _BB * _NODE)
        srcg = jnp.dot(S, nb, preferred_element_type=f32).reshape(_EP * _BB, _NODE)
        tgtg = jnp.dot(T, nb, preferred_element_type=f32).reshape(_EP * _BB, _NODE)
        comb = jnp.concatenate([srcg, tgtg, eattr_r], axis=1)
        m1 = jax.nn.relu(dot(comb, wr[b]) + wr[b + 1][...])
        msgs = (dot(m1, wr[b + 2]) + wr[b + 3][...])
        msgs_w = msgs.reshape(_EP, _BB * _NODE) * emask
        agg_w = jnp.dot(Tt, msgs_w.astype(bf16),
                        preferred_element_type=f32) / cntJ
        proj = dot(agg_w.reshape(R, _NODE), wr[b + 4]) + wr[b + 5][...]
        comb2 = jnp.concatenate([normed, proj], axis=1)
        ug = jax.nn.sigmoid(dot(comb2, wr[b + 8]) + wr[b + 9][...])
        rg = jax.nn.sigmoid(dot(comb2, wr[b + 6]) + wr[b + 7][...])
        cc = jnp.tanh(dot(jnp.concatenate([rg * normed, proj], axis=1), wr[b + 10])
                      + wr[b + 11][...])
        node = node + (1.0 - ug) * normed + ug * cc
        n2 = _lnorm(node, wr[b + 14][...], wr[b + 15][...])
        ffp = dot(n2, wr[b + 16]) + wr[b + 17][...]
        ffh = 0.5 * ffp * (1.0 + lax.erf(ffp * 0.7071067811865476))
        node = node + dot(ffh, wr[b + 18]) + wr[b + 19][...]
        eattr_r = msgs_w.reshape(_EP * _BB, _NODE)

    e = _NW_PRE + _LAYERS * _NW_LAYER
    out = _lnorm(node, wr[e][...], wr[e + 1][...])
    o8 = dot(out, wr[e + 2]) + wr[e + 3][...]
    raw = o8[:, 0:4]
    nrm = jnp.maximum(jnp.sqrt(jnp.sum(raw * raw, axis=1, keepdims=True)), 1e-8)
    m = jm_r[...].reshape(R, 1)
    rot_r[...] = ((raw / nrm) * m).reshape(_J, _BB, 4)
    conf_r[...] = (jax.nn.sigmoid(o8[:, 4:5]) * m).reshape(_J, _BB, 1)


def kernel(joint_features, topology_features, joint_mask, edge_mask,
           bone_name_tokens, source_indices, target_indices, edge_direction, params):
    f32, bf16 = jnp.float32, jnp.bfloat16
    G = _B // _BB
    BBH = _BB // 2
    jf = (joint_features.reshape(G, BBH, 2, _J, _IN)
          .transpose(0, 3, 1, 2, 4).reshape(G * _J * BBH, 2 * _IN))
    topo = (topology_features.reshape(G, BBH, 2, _J, _TOPO)
            .transpose(0, 3, 1, 2, 4).reshape(G * _J * BBH, 2 * _TOPO))
    jm = joint_mask.transpose(1, 0)[:, :, None]
    tok = (bone_name_tokens.astype(jnp.int32).reshape(G, BBH, 2, _J, _TOK)
           .transpose(0, 4, 3, 1, 2).reshape(G * _TOK * _J * BBH, 2)
           .astype(bf16))
    lanei = (jnp.arange(128, dtype=jnp.int32) % _VOCAB).astype(f32).reshape(1, 128)
    idxc = jnp.full((_EP, 8), _J, jnp.int32)
    idxc = idxc.at[:_E, 0].set(source_indices.astype(jnp.int32))
    idxc = idxc.at[:_E, 1].set(target_indices.astype(jnp.int32))
    idxc = idxc.at[:, 2].set(0).at[:_E, 2].set(edge_direction.astype(jnp.int32))
    idxr = idxc.T
    emk = jnp.zeros((_EP, 8), f32).at[:_E, 0].set(edge_mask.astype(f32))

    p = params

    def w2(d):
        return d["w"].astype(bf16)

    def b2(d):
        return d["b"].reshape(1, -1).astype(f32)

    def bdiag(w):
        i, o = w.shape
        return jnp.zeros((2 * i, 2 * o), f32).at[:i, :o].set(w).at[i:, o:].set(w)

    ip2 = bdiag(p["input_proj"]["w"])
    ip2_b = jnp.tile((p["input_proj"]["b"] + p["bone_proj"]["b"]).reshape(1, -1),
                     (1, 2))
    bcP = (jnp.zeros((2, 2 * _VOCAB), f32)
           .at[0, :_VOCAB].set(1.0).at[1, _VOCAB:].set(1.0))
    tks = [bdiag(p["char_embed"] @ p["conv_w"][k]) for k in range(3)]
    conv_b2 = jnp.tile(p["conv_b"].reshape(1, -1), (1, 2))
    bo_w = p["bone_out"]["w"]
    bo2 = (jnp.zeros((2 * (_CONV + _TOPO), 2 * _BONE), f32)
           .at[:_CONV, :_BONE].set(bo_w[:_CONV])
           .at[_CONV:2 * _CONV, _BONE:].set(bo_w[:_CONV])
           .at[2 * _CONV:2 * _CONV + _TOPO, :_BONE].set(bo_w[_CONV:])
           .at[2 * _CONV + _TOPO:, _BONE:].set(bo_w[_CONV:]))
    bo2_b = jnp.tile(p["bone_out"]["b"].reshape(1, -1), (1, 2))
    bp2 = bdiag(p["bone_proj"]["w"])

    ws = [
        ip2.astype(bf16), ip2_b,
        bcP.astype(bf16),
        tks[0].astype(bf16), tks[1].astype(bf16), tks[2].astype(bf16),
        conv_b2,
        bo2.astype(bf16), bo2_b,
        bp2.astype(bf16),
        jnp.zeros((8, _NODE), bf16).at[:2, :_EDGE].set(p["edge_dir_embed"].astype(bf16)),
    ]
    for bp in p["blocks"]:
        em1p = jnp.zeros((3 * _NODE, 4 * _EDGE), f32).at[:2 * _NODE + _EDGE].set(
            bp["edge_mlp1"]["w"])
        em2p = jnp.zeros((4 * _EDGE, _NODE), f32).at[:, :_EDGE].set(
            bp["edge_mlp2"]["w"])
        em2bp = jnp.zeros((1, _NODE), f32).at[0, :_EDGE].set(bp["edge_mlp2"]["b"])
        mpp = jnp.zeros((_NODE, _NODE), f32).at[:_EDGE].set(bp["msg_proj"]["w"])
        ws += [
            em1p.astype(bf16), b2(bp["edge_mlp1"]),
            em2p.astype(bf16), em2bp,
            mpp.astype(bf16), b2(bp["msg_proj"]),
            w2(bp["reset"]), b2(bp["reset"]),
            w2(bp["update"]), b2(bp["update"]),
            w2(bp["cand"]), b2(bp["cand"]),
            bp["norm1"]["g"].reshape(1, _NODE), bp["norm1"]["b"].reshape(1, _NODE),
            bp["norm2"]["g"].reshape(1, _NODE), bp["norm2"]["b"].reshape(1, _NODE),
            w2(bp["ffn1"]), b2(bp["ffn1"]),
            w2(bp["ffn2"]), b2(bp["ffn2"]),
        ]
    dc_w = (jnp.zeros((_NODE, 8), f32)
            .at[:, 0:4].set(p["delta"]["w"]).at[:, 4:5].set(p["conf"]["w"])).astype(bf16)
    dc_b = (jnp.zeros((1, 8), f32)
            .at[0, 0:4].set(p["delta"]["b"]).at[0, 4:5].set(p["conf"]["b"]))
    ws += [p["out_norm"]["g"].reshape(1, _NODE), p["out_norm"]["b"].reshape(1, _NODE),
           dc_w, dc_b]

    def _const(i):
        return (0, 0)

    in_specs = [
        pl.BlockSpec((_J * BBH, 2 * _IN), lambda i: (i, 0)),
        pl.BlockSpec((_J * BBH, 2 * _TOPO), lambda i: (i, 0)),
        pl.BlockSpec((_J, _BB, 1), lambda i: (0, i, 0)),
        pl.BlockSpec((_TOK * _J * BBH, 2), lambda i: (i, 0)),
        pl.BlockSpec((1, 128), _const),
        pl.BlockSpec((_EP, 8), _const),
        pl.BlockSpec((8, _EP), _const),
        pl.BlockSpec((_EP, 8), _const),
    ] + [pl.BlockSpec(w.shape, _const) for w in ws]

    rot, conf = pl.pallas_call(
        _body,
        grid=(G,),
        in_specs=in_specs,
        out_specs=[pl.BlockSpec((_J, _BB, 4), lambda i: (0, i, 0)),
                   pl.BlockSpec((_J, _BB, 1), lambda i: (0, i, 0))],
        out_shape=[jax.ShapeDtypeStruct((_J, _B, 4), f32),
                   jax.ShapeDtypeStruct((_J, _B, 1), f32)],
        compiler_params=pltpu.CompilerParams(dimension_semantics=("parallel",)),
    )(jf, topo, jm, tok, lanei, idxc, idxr, emk, *ws)
    return rot.transpose(1, 0, 2), conf.transpose(1, 0, 2)


# speedup vs baseline: 4.2744x; 1.0293x over previous
import jax
import jax.numpy as jnp
from jax import lax
from jax.experimental import pallas as pl
from jax.experimental.pallas import tpu as pltpu

_B, _J, _E, _EP = 256, 64, 126, 128
_NODE, _EDGE, _FFN, _LAYERS = 128, 32, 2048, 4
_IN, _TOPO, _VOCAB, _TOK, _CEMB, _CONV, _BONE = 9, 6, 64, 32, 32, 64, 64
_BB = 8
_NW_LAYER = 20
_NW_PRE = 11


def _lnorm(x, g, b):
    m = jnp.mean(x, axis=-1, keepdims=True)
    v = jnp.mean((x - m) ** 2, axis=-1, keepdims=True)
    return (x - m) / jnp.sqrt(v + 1e-5) * g + b


def _body(*refs):
    jf_r, topo_r, jm_r, tok_r, lane_r, idxc_r, idxr_r, emk_r = refs[:8]
    wr = refs[8:-2]
    rot_r, conf_r = refs[-2:]
    f32, bf16 = jnp.float32, jnp.bfloat16
    R = _J * _BB
    R2 = R // 2
    RT2 = R2 * _TOK

    def dot(x, w):
        return jnp.dot(x.astype(bf16), w[...], preferred_element_type=f32)

    tokp = tok_r[...]
    bc = jnp.dot(tokp, wr[2][...], preferred_element_type=f32)
    oh2 = (bc == lane_r[...]).astype(bf16)
    z2 = jnp.zeros((R2, _NODE), bf16)
    ohm = jnp.concatenate([z2, oh2[:-R2]], axis=0)
    ohp = jnp.concatenate([oh2[R2:], z2], axis=0)
    conv = (jnp.dot(ohm, wr[3][...], preferred_element_type=f32)
            + jnp.dot(oh2, wr[4][...], preferred_element_type=f32)
            + jnp.dot(ohp, wr[5][...], preferred_element_type=f32) + wr[6][...])
    pooled = jnp.max(jax.nn.relu(conv).reshape(_TOK, R2, _NODE), axis=0)
    h = jnp.concatenate([pooled, topo_r[...]], axis=1)
    bone = jax.nn.relu(dot(h, wr[7]) + wr[8][...])
    node2 = dot(jf_r[...], wr[0]) + wr[1][...] + dot(bone, wr[9])
    node = node2.reshape(R, _NODE)

    idxc = idxc_r[...]
    idxr = idxr_r[...]
    iotaJ = lax.broadcasted_iota(jnp.int32, (_EP, _J), 1)
    S = (idxc[:, 0:1] == iotaJ).astype(bf16)
    T = (idxc[:, 1:2] == iotaJ).astype(bf16)
    Tt = (idxr[1:2, :] ==
          lax.broadcasted_iota(jnp.int32, (_J, _EP), 0)).astype(bf16)
    D = (idxc[:, 2:3] == lax.broadcasted_iota(jnp.int32, (_EP, 8), 1)).astype(bf16)
    eattr0 = jnp.dot(D, wr[10][...], preferred_element_type=f32)
    eattr_r = jnp.concatenate([eattr0] * _BB, axis=1).reshape(_EP * _BB, _NODE)
    emask = emk_r[...][:, 0:1]
    cntJ = jnp.maximum(jnp.sum(Tt.astype(f32), axis=1, keepdims=True), 1.0)

    for l in range(_LAYERS):
        em1 = wr[11][pl.ds(l * 384, 384), :]
        em2 = wr[12][pl.ds(l * _NODE, _NODE), :]
        mp = wr[13][pl.ds(l * _NODE, _NODE), :]
        ru = wr[14][pl.ds(l * 256, 256), :]
        cw = wr[15][pl.ds(l * 256, 256), :]
        f1 = wr[16][pl.ds(l * _NODE, _NODE), :]
        f2 = wr[17][pl.ds(l * _FFN, _FFN), :]
        f1b = wr[19][pl.ds(l, 1), :]
        b0 = l * 11

        def brow(k, _b0=b0):
            return wr[18][pl.ds(_b0 + k, 1), :]

        normed = _lnorm(node, brow(6), brow(7))
        nb = normed.astype(bf16).reshape(_J, _BB * _NODE)
        srcg = jnp.dot(S, nb, preferred_element_type=f32).reshape(_EP * _BB, _NODE)
        tgtg = jnp.dot(T, nb, preferred_element_type=f32).reshape(_EP * _BB, _NODE)
        comb = jnp.concatenate([srcg, tgtg, eattr_r], axis=1)
        m1 = jax.nn.relu(dot(comb, em1) + brow(0))
        msgs = dot(m1, em2) + brow(1)
        msgs_w = msgs.reshape(_EP, _BB * _NODE) * emask
        agg_w = jnp.dot(Tt, msgs_w.astype(bf16),
                        preferred_element_type=f32) / cntJ
        proj = dot(agg_w.reshape(R, _NODE), mp) + brow(2)
        comb2 = jnp.concatenate([normed, proj], axis=1)
        g2 = dot(comb2, ru)
        rg = jax.nn.sigmoid(g2[:, :_NODE] + brow(3))
        ug = jax.nn.sigmoid(g2[:, _NODE:] + brow(4))
        cc = jnp.tanh(dot(jnp.concatenate([rg * normed, proj], axis=1), cw)
                      + brow(5))
        node = node + (1.0 - ug) * normed + ug * cc
        n2 = _lnorm(node, brow(8), brow(9))
        ffp = dot(n2, f1) + f1b
        ffh = 0.5 * ffp * (1.0 + lax.erf(ffp * 0.7071067811865476))
        node = node + dot(ffh, f2) + brow(10)
        eattr_r = msgs_w.reshape(_EP * _BB, _NODE)

    e = 20
    out = _lnorm(node, wr[e][...], wr[e + 1][...])
    o8 = dot(out, wr[e + 2]) + wr[e + 3][...]
    raw = o8[:, 0:4]
    nrm = jnp.maximum(jnp.sqrt(jnp.sum(raw * raw, axis=1, keepdims=True)), 1e-8)
    m = jm_r[...].reshape(R, 1)
    rot_r[...] = ((raw / nrm) * m).reshape(_J, _BB, 4)
    conf_r[...] = (jax.nn.sigmoid(o8[:, 4:5]) * m).reshape(_J, _BB, 1)


def kernel(joint_features, topology_features, joint_mask, edge_mask,
           bone_name_tokens, source_indices, target_indices, edge_direction, params):
    f32, bf16 = jnp.float32, jnp.bfloat16
    G = _B // _BB
    BBH = _BB // 2
    jf = (joint_features.reshape(G, BBH, 2, _J, _IN)
          .transpose(0, 3, 1, 2, 4).reshape(G * _J * BBH, 2 * _IN))
    topo = (topology_features.reshape(G, BBH, 2, _J, _TOPO)
            .transpose(0, 3, 1, 2, 4).reshape(G * _J * BBH, 2 * _TOPO))
    jm = joint_mask.transpose(1, 0)[:, :, None]
    tok = (bone_name_tokens.astype(jnp.int32).reshape(G, BBH, 2, _J, _TOK)
           .transpose(0, 4, 3, 1, 2).reshape(G * _TOK * _J * BBH, 2)
           .astype(bf16))
    lanei = (jnp.arange(128, dtype=jnp.int32) % _VOCAB).astype(f32).reshape(1, 128)
    idxc = jnp.full((_EP, 8), _J, jnp.int32)
    idxc = idxc.at[:_E, 0].set(source_indices.astype(jnp.int32))
    idxc = idxc.at[:_E, 1].set(target_indices.astype(jnp.int32))
    idxc = idxc.at[:, 2].set(0).at[:_E, 2].set(edge_direction.astype(jnp.int32))
    idxr = idxc.T
    emk = jnp.zeros((_EP, 8), f32).at[:_E, 0].set(edge_mask.astype(f32))

    p = params

    def w2(d):
        return d["w"].astype(bf16)

    def b2(d):
        return d["b"].reshape(1, -1).astype(f32)

    def bdiag(w):
        i, o = w.shape
        return jnp.zeros((2 * i, 2 * o), f32).at[:i, :o].set(w).at[i:, o:].set(w)

    ip2 = bdiag(p["input_proj"]["w"])
    ip2_b = jnp.tile((p["input_proj"]["b"] + p["bone_proj"]["b"]).reshape(1, -1),
                     (1, 2))
    bcP = (jnp.zeros((2, 2 * _VOCAB), f32)
           .at[0, :_VOCAB].set(1.0).at[1, _VOCAB:].set(1.0))
    tks = [bdiag(p["char_embed"] @ p["conv_w"][k]) for k in range(3)]
    conv_b2 = jnp.tile(p["conv_b"].reshape(1, -1), (1, 2))
    bo_w = p["bone_out"]["w"]
    bo2 = (jnp.zeros((2 * (_CONV + _TOPO), 2 * _BONE), f32)
           .at[:_CONV, :_BONE].set(bo_w[:_CONV])
           .at[_CONV:2 * _CONV, _BONE:].set(bo_w[:_CONV])
           .at[2 * _CONV:2 * _CONV + _TOPO, :_BONE].set(bo_w[_CONV:])
           .at[2 * _CONV + _TOPO:, _BONE:].set(bo_w[_CONV:]))
    bo2_b = jnp.tile(p["bone_out"]["b"].reshape(1, -1), (1, 2))
    bp2 = bdiag(p["bone_proj"]["w"])

    ws = [
        ip2.astype(bf16), ip2_b,
        bcP.astype(bf16),
        tks[0].astype(bf16), tks[1].astype(bf16), tks[2].astype(bf16),
        conv_b2,
        bo2.astype(bf16), bo2_b,
        bp2.astype(bf16),
        jnp.zeros((8, _NODE), bf16).at[:2, :_EDGE].set(p["edge_dir_embed"].astype(bf16)),
    ]
    blocks = p["blocks"]
    em1s = jnp.pad(
        jnp.concatenate([bp["edge_mlp1"]["w"] for bp in blocks], axis=0)
        .reshape(_LAYERS, 2 * _NODE + _EDGE, _NODE),
        ((0, 0), (0, 3 * _NODE - 2 * _NODE - _EDGE), (0, 0))
    ).reshape(_LAYERS * 3 * _NODE, _NODE)
    em2s = jnp.pad(
        jnp.concatenate([bp["edge_mlp2"]["w"] for bp in blocks], axis=0)
        .reshape(_LAYERS, _NODE, _EDGE),
        ((0, 0), (0, 0), (0, _NODE - _EDGE))
    ).reshape(_LAYERS * _NODE, _NODE)
    mps = jnp.pad(
        jnp.concatenate([bp["msg_proj"]["w"] for bp in blocks], axis=0)
        .reshape(_LAYERS, _EDGE, _NODE),
        ((0, 0), (0, _NODE - _EDGE), (0, 0))
    ).reshape(_LAYERS * _NODE, _NODE)
    rus = jnp.concatenate(
        [jnp.concatenate([bp["reset"]["w"], bp["update"]["w"]], axis=1)
         for bp in blocks], axis=0)
    cs = jnp.concatenate([bp["cand"]["w"] for bp in blocks], axis=0)
    f1s = jnp.concatenate([bp["ffn1"]["w"] for bp in blocks], axis=0)
    f2s = jnp.concatenate([bp["ffn2"]["w"] for bp in blocks], axis=0)
    rows = []
    for bp in blocks:
        rows += [
            bp["edge_mlp1"]["b"],
            jnp.pad(bp["edge_mlp2"]["b"], (0, _NODE - _EDGE)),
            bp["msg_proj"]["b"], bp["reset"]["b"], bp["update"]["b"],
            bp["cand"]["b"], bp["norm1"]["g"], bp["norm1"]["b"],
            bp["norm2"]["g"], bp["norm2"]["b"], bp["ffn2"]["b"],
        ]
    brows = jnp.stack(rows, axis=0)
    f1bs = jnp.stack([bp["ffn1"]["b"] for bp in blocks], axis=0)
    ws += [em1s.astype(bf16), em2s.astype(bf16), mps.astype(bf16),
           rus.astype(bf16), cs.astype(bf16), f1s.astype(bf16),
           f2s.astype(bf16), brows, f1bs]
    dc_w = (jnp.zeros((_NODE, 8), f32)
            .at[:, 0:4].set(p["delta"]["w"]).at[:, 4:5].set(p["conf"]["w"])).astype(bf16)
    dc_b = (jnp.zeros((1, 8), f32)
            .at[0, 0:4].set(p["delta"]["b"]).at[0, 4:5].set(p["conf"]["b"]))
    ws += [p["out_norm"]["g"].reshape(1, _NODE), p["out_norm"]["b"].reshape(1, _NODE),
           dc_w, dc_b]

    def _const(i):
        return (0, 0)

    in_specs = [
        pl.BlockSpec((_J * BBH, 2 * _IN), lambda i: (i, 0)),
        pl.BlockSpec((_J * BBH, 2 * _TOPO), lambda i: (i, 0)),
        pl.BlockSpec((_J, _BB, 1), lambda i: (0, i, 0)),
        pl.BlockSpec((_TOK * _J * BBH, 2), lambda i: (i, 0)),
        pl.BlockSpec((1, 128), _const),
        pl.BlockSpec((_EP, 8), _const),
        pl.BlockSpec((8, _EP), _const),
        pl.BlockSpec((_EP, 8), _const),
    ] + [pl.BlockSpec(w.shape, _const) for w in ws]

    rot, conf = pl.pallas_call(
        _body,
        grid=(G,),
        in_specs=in_specs,
        out_specs=[pl.BlockSpec((_J, _BB, 4), lambda i: (0, i, 0)),
                   pl.BlockSpec((_J, _BB, 1), lambda i: (0, i, 0))],
        out_shape=[jax.ShapeDtypeStruct((_J, _B, 4), f32),
                   jax.ShapeDtypeStruct((_J, _B, 1), f32)],
        compiler_params=pltpu.CompilerParams(dimension_semantics=("parallel",)),
    )(jf, topo, jm, tok, lanei, idxc, idxr, emk, *ws)
    return rot.transpose(1, 0, 2), conf.transpose(1, 0, 2)
